# baseline (speedup 1.0000x reference)
import functools

import jax
import jax.numpy as jnp
from jax import lax
from jax.experimental import pallas as pl
from jax.experimental.pallas import tpu as pltpu

N = 4096
D_PAD = 384
PCOLS = 512
ROWS_BLK = 128


def _p_body(b_ref, u_ref, v_ref, p_ref, vb_ref):
    j = pl.program_id(0)

    def body(i, carry):
        idx = b_ref[j * PCOLS + i]
        vb_ref[pl.ds(i, 1), :] = v_ref[pl.ds(idx, 1), :]
        return carry

    lax.fori_loop(0, PCOLS, body, 0)
    acc = lax.dot_general(
        u_ref[...], vb_ref[...], (((1,), (1,)), ((), ())),
        preferred_element_type=jnp.float32)
    p_ref[...] = jax.nn.sigmoid(acc)


def _gather_body(a_ref, p_ref, o_ref):
    j = pl.program_id(0)

    def body(i, carry):
        idx = a_ref[j * ROWS_BLK + i]
        o_ref[pl.ds(i, 1), :] = p_ref[pl.ds(idx, 1), :]
        return carry

    lax.fori_loop(0, ROWS_BLK, body, 0)


def kernel(X, U, V):
    a = X[:, 0]
    b = X[:, 1]
    vocab, d = U.shape
    u_pad = jnp.pad(U, ((0, 0), (0, D_PAD - d)))
    v_pad = jnp.pad(V, ((0, 0), (0, D_PAD - d)))

    p = pl.pallas_call(
        _p_body,
        grid_spec=pltpu.PrefetchScalarGridSpec(
            num_scalar_prefetch=1,
            grid=(N // PCOLS,),
            in_specs=[
                pl.BlockSpec((vocab, D_PAD), lambda j, b_: (0, 0)),
                pl.BlockSpec((vocab, D_PAD), lambda j, b_: (0, 0)),
            ],
            out_specs=pl.BlockSpec((vocab, PCOLS), lambda j, b_: (0, j)),
            scratch_shapes=[pltpu.VMEM((PCOLS, D_PAD), jnp.float32)],
        ),
        out_shape=jax.ShapeDtypeStruct((vocab, N), jnp.float32),
    )(b, u_pad, v_pad)

    out = pl.pallas_call(
        _gather_body,
        grid_spec=pltpu.PrefetchScalarGridSpec(
            num_scalar_prefetch=1,
            grid=(N // ROWS_BLK,),
            in_specs=[pl.BlockSpec((vocab, N), lambda j, a_: (0, 0))],
            out_specs=pl.BlockSpec((ROWS_BLK, N), lambda j, a_: (j, 0)),
        ),
        out_shape=jax.ShapeDtypeStruct((N, N), jnp.float32),
    )(a, p)
    return out


# speedup vs baseline: 1.2800x; 1.2800x over previous
import functools

import jax
import jax.numpy as jnp
from jax import lax
from jax.experimental import pallas as pl
from jax.experimental.pallas import tpu as pltpu
from jax.experimental.pallas import tpu_sc as plsc

N = 4096
D_PAD = 384
PCOLS = 512
NC, NS = 2, 16
NW = NC * NS


def _sc_gather_small_body(table, idx, out, idx_v, buf_v, gsem):
    wid = lax.axis_index("s") * NC + lax.axis_index("c")
    rows = N // NW
    base = wid * rows
    pltpu.sync_copy(idx.at[pl.ds(base, rows)], idx_v)
    pltpu.async_copy(table.at[idx_v], buf_v, gsem).wait()
    pltpu.sync_copy(buf_v, out.at[pl.ds(base, rows)])


def _sc_gather_big_body(table, idx, out, idx_v, buf_v, gsem, ssem):
    wid = lax.axis_index("s") * NC + lax.axis_index("c")
    rows = N // NW
    rpc = 8
    nch = rows // rpc
    nbuf = 3
    base = wid * rows
    pltpu.sync_copy(idx.at[pl.ds(base, rows)], idx_v)

    def fire(c):
        return pltpu.async_copy(
            table.at[idx_v.at[pl.ds(c * rpc, rpc)]], buf_v.at[c % nbuf], gsem)

    def put(c):
        return pltpu.async_copy(
            buf_v.at[c % nbuf], out.at[pl.ds(base + c * rpc, rpc)], ssem)

    g = {}
    s = {}
    for c in range(nbuf - 1):
        g[c] = fire(c)
    for c in range(nch):
        la = c + nbuf - 1
        if la < nch:
            if la >= nbuf:
                s[la - nbuf].wait()
            g[la] = fire(la)
        g[c].wait()
        s[c] = put(c)
    for c in range(max(0, nch - nbuf), nch):
        s[c].wait()


def _matmul_body(u_ref, vb_ref, p_ref):
    acc = lax.dot_general(
        u_ref[...], vb_ref[...], (((1,), (1,)), ((), ())),
        preferred_element_type=jnp.float32)
    p_ref[...] = jax.nn.sigmoid(acc)


def kernel(X, U, V):
    a = X[:, 0]
    b = X[:, 1]
    vocab, d = U.shape
    u_pad = jnp.pad(U, ((0, 0), (0, D_PAD - d)))
    v_pad = jnp.pad(V, ((0, 0), (0, D_PAD - d)))

    mesh = plsc.VectorSubcoreMesh(core_axis_name="c", subcore_axis_name="s")
    rows = N // NW

    vb = pl.kernel(
        _sc_gather_small_body,
        out_type=jax.ShapeDtypeStruct((N, D_PAD), jnp.float32),
        mesh=mesh,
        scratch_types=[
            pltpu.VMEM((rows,), jnp.int32),
            pltpu.VMEM((rows, D_PAD), jnp.float32),
            pltpu.SemaphoreType.DMA,
        ],
    )(v_pad, b)

    p = pl.pallas_call(
        _matmul_body,
        grid=(N // PCOLS,),
        in_specs=[
            pl.BlockSpec((vocab, D_PAD), lambda j: (0, 0)),
            pl.BlockSpec((PCOLS, D_PAD), lambda j: (j, 0)),
        ],
        out_specs=pl.BlockSpec((vocab, PCOLS), lambda j: (0, j)),
        out_shape=jax.ShapeDtypeStruct((vocab, N), jnp.float32),
    )(u_pad, vb)

    out = pl.kernel(
        _sc_gather_big_body,
        out_type=jax.ShapeDtypeStruct((N, N), jnp.float32),
        mesh=mesh,
        scratch_types=[
            pltpu.VMEM((rows,), jnp.int32),
            pltpu.VMEM((3, 8, N), jnp.float32),
            pltpu.SemaphoreType.DMA,
            pltpu.SemaphoreType.DMA,
        ],
    )(p, a)
    return out
